# baseline (device time: 53777 ns/iter reference)
import jax
import jax.numpy as jnp
from jax import lax
from jax.experimental import pallas as pl
from jax.experimental.pallas import tpu as pltpu

N_Z = 4


def kernel(partial, resid, gamma):
    _, m, d = partial.shape
    mh = m // 2
    mc = mh // N_Z
    gamma2 = gamma.reshape(1, d)

    def body(p_ref, r_ref, g_ref, out_ref,
             rs_p_out, rs_m_out, rs_p_in, rs_m_in,
             norm_buf, ag_p_in, ag_m_in, xg_own, xg_p, xg_m, acc,
             rs_p_send_sems, rs_p_recv_sems, rs_m_send_sems, rs_m_recv_sems,
             ag_p_send_sems, ag_p_recv_sems, ag_m_send_sems, ag_m_recv_sems,
             x_own_send_sem, x_own_recv_sem,
             xg_p_send_sems, xg_p_recv_sems, xg_m_send_sems, xg_m_recv_sems):
        my_x = lax.axis_index("x")
        my_y = lax.axis_index("y")
        my_z = lax.axis_index("z")
        xn = 1 - my_x
        my_base = my_x * mh
        nb_base = xn * mh
        has_l = my_z > 0
        has_r = my_z < N_Z - 1
        left = jnp.maximum(my_z - 1, 0)
        right = jnp.minimum(my_z + 1, N_Z - 1)

        barrier_sem = pltpu.get_barrier_semaphore()

        @pl.when(has_l)
        def _():
            pl.semaphore_signal(barrier_sem, inc=1,
                                device_id=(my_x, my_y, left),
                                device_id_type=pl.DeviceIdType.MESH)

        @pl.when(has_r)
        def _():
            pl.semaphore_signal(barrier_sem, inc=1,
                                device_id=(my_x, my_y, right),
                                device_id_type=pl.DeviceIdType.MESH)

        pl.semaphore_signal(barrier_sem, inc=1, device_id=(xn, my_y, my_z),
                            device_id_type=pl.DeviceIdType.MESH)
        pl.semaphore_wait(barrier_sem, 1)

        @pl.when(has_l)
        def _():
            pl.semaphore_wait(barrier_sem, 1)

        @pl.when(has_r)
        def _():
            pl.semaphore_wait(barrier_sem, 1)

        def my_chunk(c):
            return p_ref[0, pl.ds(my_base + c * mc, mc), :]

        def rdma(src, dst, send_sem, recv_sem, dev):
            return pltpu.make_async_remote_copy(
                src_ref=src, dst_ref=dst, send_sem=send_sem,
                recv_sem=recv_sem, device_id=dev,
                device_id_type=pl.DeviceIdType.MESH)

        r_dev = (my_x, my_y, right)
        l_dev = (my_x, my_y, left)
        drains = []

        rs_p = [rdma(rs_p_out.at[k - 1], rs_p_in.at[k - 1],
                     rs_p_send_sems.at[k - 1], rs_p_recv_sems.at[k - 1],
                     r_dev) for k in range(1, N_Z)]
        for k in range(1, N_Z):
            pred = my_z + k <= N_Z - 1

            @pl.when(pred)
            def _(k=k):
                rs_p_out[k - 1] = my_chunk(my_z + k).astype(jnp.bfloat16)

            if k <= N_Z - 2:
                @pl.when(pred & has_l)
                def _(k=k):
                    rs_p[k].wait_recv()
                    rs_p_out[k - 1] = (
                        rs_p_out[k - 1].astype(jnp.float32)
                        + rs_p_in[k].astype(jnp.float32)).astype(jnp.bfloat16)

            @pl.when(pred)
            def _(k=k):
                rs_p[k - 1].start()
            drains.append((pred, rs_p[k - 1]))

        rs_m = [rdma(rs_m_out.at[k - 1], rs_m_in.at[k - 1],
                     rs_m_send_sems.at[k - 1], rs_m_recv_sems.at[k - 1],
                     l_dev) for k in range(1, N_Z)]
        for k in range(1, N_Z):
            pred = my_z - k >= 0

            @pl.when(pred)
            def _(k=k):
                rs_m_out[k - 1] = my_chunk(my_z - k).astype(jnp.bfloat16)

            if k <= N_Z - 2:
                @pl.when(pred & has_r)
                def _(k=k):
                    rs_m[k].wait_recv()
                    rs_m_out[k - 1] = (
                        rs_m_out[k - 1].astype(jnp.float32)
                        + rs_m_in[k].astype(jnp.float32)).astype(jnp.bfloat16)

            @pl.when(pred)
            def _(k=k):
                rs_m[k - 1].start()
            drains.append((pred, rs_m[k - 1]))

        acc[...] = my_chunk(my_z)

        @pl.when(has_l)
        def _():
            rs_p[0].wait_recv()
            acc[...] = acc[...] + rs_p_in[0].astype(jnp.float32)

        @pl.when(has_r)
        def _():
            rs_m[0].wait_recv()
            acc[...] = acc[...] + rs_m_in[0].astype(jnp.float32)

        y = acc[...] + r_ref[pl.ds(my_base + my_z * mc, mc), :]
        rms = jnp.sqrt(jnp.mean(y * y, axis=-1, keepdims=True) + 1e-6)
        norm = (y / rms) * g_ref[...]
        norm_buf[...] = norm.astype(jnp.bfloat16)
        out_ref[pl.ds(my_base + my_z * mc, mc), :] = norm

        ag_p = [rdma(norm_buf if k == 0 else ag_p_in.at[k - 1],
                     ag_p_in.at[k] if k < N_Z - 1 else ag_p_in.at[0],
                     ag_p_send_sems.at[k], ag_p_recv_sems.at[k], r_dev)
                for k in range(N_Z - 1)]
        ag_m = [rdma(norm_buf if k == 0 else ag_m_in.at[k - 1],
                     ag_m_in.at[k] if k < N_Z - 1 else ag_m_in.at[0],
                     ag_m_send_sems.at[k], ag_m_recv_sems.at[k], l_dev)
                for k in range(N_Z - 1)]
        x_own = rdma(norm_buf, xg_own, x_own_send_sem.at[0], x_own_recv_sem.at[0],
                     (xn, my_y, my_z))
        x_own.start()
        drains.append((None, x_own))

        @pl.when(has_r)
        def _():
            ag_p[0].start()
        drains.append((has_r, ag_p[0]))

        @pl.when(has_l)
        def _():
            ag_m[0].start()
        drains.append((has_l, ag_m[0]))

        x_p = [rdma(ag_p_in.at[k - 1], xg_p.at[k - 1],
                    xg_p_send_sems.at[k - 1], xg_p_recv_sems.at[k - 1],
                    (xn, my_y, my_z)) for k in range(1, N_Z)]
        x_m = [rdma(ag_m_in.at[k - 1], xg_m.at[k - 1],
                    xg_m_send_sems.at[k - 1], xg_m_recv_sems.at[k - 1],
                    (xn, my_y, my_z)) for k in range(1, N_Z)]

        for k in range(1, N_Z):
            pred_p = my_z - k >= 0

            @pl.when(pred_p)
            def _(k=k):
                ag_p[k - 1].wait_recv()

            if k < N_Z - 1:
                @pl.when(pred_p & has_r)
                def _(k=k):
                    ag_p[k].start()
                drains.append((pred_p & has_r, ag_p[k]))

            @pl.when(pred_p)
            def _(k=k):
                x_p[k - 1].start()
                c = my_z - k
                out_ref[pl.ds(my_base + c * mc, mc), :] = (
                    ag_p_in[k - 1].astype(jnp.float32))
            drains.append((pred_p, x_p[k - 1]))

            pred_m = my_z + k <= N_Z - 1

            @pl.when(pred_m)
            def _(k=k):
                ag_m[k - 1].wait_recv()

            if k < N_Z - 1:
                @pl.when(pred_m & has_l)
                def _(k=k):
                    ag_m[k].start()
                drains.append((pred_m & has_l, ag_m[k]))

            @pl.when(pred_m)
            def _(k=k):
                x_m[k - 1].start()
                c = my_z + k
                out_ref[pl.ds(my_base + c * mc, mc), :] = (
                    ag_m_in[k - 1].astype(jnp.float32))
            drains.append((pred_m, x_m[k - 1]))

        x_own.wait_recv()
        out_ref[pl.ds(nb_base + my_z * mc, mc), :] = xg_own[...].astype(jnp.float32)
        for k in range(1, N_Z):
            @pl.when(my_z - k >= 0)
            def _(k=k):
                x_p[k - 1].wait_recv()
                c = my_z - k
                out_ref[pl.ds(nb_base + c * mc, mc), :] = (
                    xg_p[k - 1].astype(jnp.float32))

            @pl.when(my_z + k <= N_Z - 1)
            def _(k=k):
                x_m[k - 1].wait_recv()
                c = my_z + k
                out_ref[pl.ds(nb_base + c * mc, mc), :] = (
                    xg_m[k - 1].astype(jnp.float32))

        for pred, r in drains:
            if pred is None:
                r.wait_send()
            else:
                @pl.when(pred)
                def _(r=r):
                    r.wait_send()

    return pl.pallas_call(
        body,
        out_shape=jax.ShapeDtypeStruct((m, d), jnp.float32),
        in_specs=[
            pl.BlockSpec(memory_space=pltpu.VMEM),
            pl.BlockSpec(memory_space=pltpu.VMEM),
            pl.BlockSpec(memory_space=pltpu.VMEM),
        ],
        out_specs=pl.BlockSpec(memory_space=pltpu.VMEM),
        scratch_shapes=[
            pltpu.VMEM((N_Z - 1, mc, d), jnp.bfloat16),
            pltpu.VMEM((N_Z - 1, mc, d), jnp.bfloat16),
            pltpu.VMEM((N_Z - 1, mc, d), jnp.bfloat16),
            pltpu.VMEM((N_Z - 1, mc, d), jnp.bfloat16),
            pltpu.VMEM((mc, d), jnp.bfloat16),
            pltpu.VMEM((N_Z - 1, mc, d), jnp.bfloat16),
            pltpu.VMEM((N_Z - 1, mc, d), jnp.bfloat16),
            pltpu.VMEM((mc, d), jnp.bfloat16),
            pltpu.VMEM((N_Z - 1, mc, d), jnp.bfloat16),
            pltpu.VMEM((N_Z - 1, mc, d), jnp.bfloat16),
            pltpu.VMEM((mc, d), jnp.float32),
            pltpu.SemaphoreType.DMA((N_Z - 1,)),
            pltpu.SemaphoreType.DMA((N_Z - 1,)),
            pltpu.SemaphoreType.DMA((N_Z - 1,)),
            pltpu.SemaphoreType.DMA((N_Z - 1,)),
            pltpu.SemaphoreType.DMA((N_Z - 1,)),
            pltpu.SemaphoreType.DMA((N_Z - 1,)),
            pltpu.SemaphoreType.DMA((N_Z - 1,)),
            pltpu.SemaphoreType.DMA((N_Z - 1,)),
            pltpu.SemaphoreType.DMA((1,)),
            pltpu.SemaphoreType.DMA((1,)),
            pltpu.SemaphoreType.DMA((N_Z - 1,)),
            pltpu.SemaphoreType.DMA((N_Z - 1,)),
            pltpu.SemaphoreType.DMA((N_Z - 1,)),
            pltpu.SemaphoreType.DMA((N_Z - 1,)),
        ],
        compiler_params=pltpu.CompilerParams(collective_id=0),
    )(partial, resid, gamma2)


# device time: 44875 ns/iter; 1.1984x vs baseline; 1.1984x over previous
import jax
import jax.numpy as jnp
from jax import lax
from jax.experimental import pallas as pl
from jax.experimental.pallas import tpu as pltpu

N_Z = 4


def kernel(partial, resid, gamma):
    _, m, d = partial.shape
    mh = m // 2
    mc = mh // N_Z
    gamma2 = gamma.reshape(1, d)

    def body(p_ref, r_ref, g_ref, out_ref,
             rs_send, rs_recv, ag_buf, xg_buf,
             rs_send_sems, rs_recv_sems, ag_send_sems, ag_recv_sems,
             x_send_sems, x_recv_sems):
        my_x = lax.axis_index("x")
        my_y = lax.axis_index("y")
        my_z = lax.axis_index("z")
        left = (my_z - 1) % N_Z
        right = (my_z + 1) % N_Z
        xn = 1 - my_x
        my_base = my_x * mh
        nb_base = xn * mh

        barrier_sem = pltpu.get_barrier_semaphore()
        for dev in [(my_x, my_y, left), (my_x, my_y, right), (xn, my_y, my_z)]:
            pl.semaphore_signal(
                barrier_sem, inc=1,
                device_id=dev, device_id_type=pl.DeviceIdType.MESH,
            )
        pl.semaphore_wait(barrier_sem, 3)

        def my_chunk(c):
            return p_ref[0, pl.ds(my_base + c * mc, mc), :]

        own = (my_z + 1) % N_Z
        drains = []

        base = None
        for s in range(N_Z - 1):
            send_c = (my_z - s) % N_Z
            if s == 0:
                val = my_chunk(send_c).astype(jnp.bfloat16)
            else:
                val = (rs_recv[s - 1].astype(jnp.float32)
                       + my_chunk(send_c)).astype(jnp.bfloat16)
            rs_send[s] = val
            rdma = pltpu.make_async_remote_copy(
                src_ref=rs_send.at[s],
                dst_ref=rs_recv.at[s],
                send_sem=rs_send_sems.at[s],
                recv_sem=rs_recv_sems.at[s],
                device_id=(my_x, my_y, right),
                device_id_type=pl.DeviceIdType.MESH,
            )
            rdma.start()
            drains.append(rdma)
            if s == 0:
                base = my_chunk(own) + r_ref[pl.ds(my_base + own * mc, mc), :]
            rdma.wait_recv()

        y = base + rs_recv[N_Z - 2].astype(jnp.float32)
        rms = jnp.sqrt(jnp.mean(y * y, axis=-1, keepdims=True) + 1e-6)
        norm = (y / rms) * g_ref[...]
        ag_buf[0] = norm.astype(jnp.bfloat16)

        def x_push(slot):
            rdma = pltpu.make_async_remote_copy(
                src_ref=ag_buf.at[slot],
                dst_ref=xg_buf.at[slot],
                send_sem=x_send_sems.at[slot],
                recv_sem=x_recv_sems.at[slot],
                device_id=(xn, my_y, my_z),
                device_id_type=pl.DeviceIdType.MESH,
            )
            rdma.start()
            drains.append(rdma)
            return rdma

        ag_rdmas = []
        x_rdmas = []
        for t in range(N_Z - 1):
            rdma = pltpu.make_async_remote_copy(
                src_ref=ag_buf.at[t],
                dst_ref=ag_buf.at[t + 1],
                send_sem=ag_send_sems.at[t],
                recv_sem=ag_recv_sems.at[t],
                device_id=(my_x, my_y, right),
                device_id_type=pl.DeviceIdType.MESH,
            )
            rdma.start()
            drains.append(rdma)
            ag_rdmas.append(rdma)
            x_rdmas.append(x_push(t))
            if t == 0:
                out_ref[pl.ds(my_base + own * mc, mc), :] = norm
            else:
                c = (own - t) % N_Z
                out_ref[pl.ds(my_base + c * mc, mc), :] = (
                    ag_buf[t].astype(jnp.float32))
            rdma.wait_recv()

        x_rdmas.append(x_push(N_Z - 1))
        c_last = (own - (N_Z - 1)) % N_Z
        out_ref[pl.ds(my_base + c_last * mc, mc), :] = (
            ag_buf[N_Z - 1].astype(jnp.float32))

        for k in range(N_Z):
            x_rdmas[k].wait_recv()
            c = (own - k) % N_Z
            out_ref[pl.ds(nb_base + c * mc, mc), :] = (
                xg_buf[k].astype(jnp.float32))

        for rdma in drains:
            rdma.wait_send()

    return pl.pallas_call(
        body,
        out_shape=jax.ShapeDtypeStruct((m, d), jnp.float32),
        in_specs=[
            pl.BlockSpec(memory_space=pltpu.VMEM),
            pl.BlockSpec(memory_space=pltpu.VMEM),
            pl.BlockSpec(memory_space=pltpu.VMEM),
        ],
        out_specs=pl.BlockSpec(memory_space=pltpu.VMEM),
        scratch_shapes=[
            pltpu.VMEM((N_Z - 1, mc, d), jnp.bfloat16),
            pltpu.VMEM((N_Z - 1, mc, d), jnp.bfloat16),
            pltpu.VMEM((N_Z, mc, d), jnp.bfloat16),
            pltpu.VMEM((N_Z, mc, d), jnp.bfloat16),
            pltpu.SemaphoreType.DMA((N_Z - 1,)),
            pltpu.SemaphoreType.DMA((N_Z - 1,)),
            pltpu.SemaphoreType.DMA((N_Z - 1,)),
            pltpu.SemaphoreType.DMA((N_Z - 1,)),
            pltpu.SemaphoreType.DMA((N_Z,)),
            pltpu.SemaphoreType.DMA((N_Z,)),
        ],
        compiler_params=pltpu.CompilerParams(collective_id=0),
    )(partial, resid, gamma2)


# device time: 43162 ns/iter; 1.2459x vs baseline; 1.0397x over previous
import jax
import jax.numpy as jnp
from jax import lax
from jax.experimental import pallas as pl
from jax.experimental.pallas import tpu as pltpu

N_Z = 4
N_Y = 4


def kernel(partial, resid, gamma):
    _, m, d = partial.shape
    mh = m // 2
    mg = mh // N_Y
    mz = mg // N_Z
    gamma2 = gamma.reshape(1, d)

    def body(p_ref, r_ref, g_ref, out_ref,
             rs_send, rs_recv, grp_buf, yg_buf, xg_buf,
             rs_send_sems, rs_recv_sems, ag_send_sems, ag_recv_sems,
             yg_send_sems, yg_recv_sems, x_send_sems, x_recv_sems):
        my_x = lax.axis_index("x")
        my_y = lax.axis_index("y")
        my_z = lax.axis_index("z")
        xn = 1 - my_x
        gb = my_x * mh + my_y * mg

        barrier_sem = pltpu.get_barrier_semaphore()
        for k in range(1, N_Z):
            pl.semaphore_signal(barrier_sem, inc=1,
                                device_id=(my_x, my_y, (my_z + k) % N_Z),
                                device_id_type=pl.DeviceIdType.MESH)
        for k in range(1, N_Y):
            pl.semaphore_signal(barrier_sem, inc=1,
                                device_id=(my_x, (my_y + k) % N_Y, my_z),
                                device_id_type=pl.DeviceIdType.MESH)
        pl.semaphore_signal(barrier_sem, inc=1, device_id=(xn, my_y, my_z),
                            device_id_type=pl.DeviceIdType.MESH)
        pl.semaphore_wait(barrier_sem, 7)

        def my_slice(c):
            return p_ref[0, pl.ds(gb + c * mz, mz), :]

        drains = []

        rs_rdmas = []
        for k in range(1, N_Z):
            r = (my_z - k) % N_Z
            rs_send[k - 1] = my_slice(r).astype(jnp.bfloat16)
            rdma = pltpu.make_async_remote_copy(
                src_ref=rs_send.at[k - 1],
                dst_ref=rs_recv.at[k - 1],
                send_sem=rs_send_sems.at[k - 1],
                recv_sem=rs_recv_sems.at[k - 1],
                device_id=(my_x, my_y, r),
                device_id_type=pl.DeviceIdType.MESH,
            )
            rdma.start()
            drains.append(rdma)
            rs_rdmas.append(rdma)

        base = my_slice(my_z) + r_ref[pl.ds(gb + my_z * mz, mz), :]
        for j in range(N_Z - 1):
            rs_rdmas[j].wait_recv()
            base = base + rs_recv[j].astype(jnp.float32)

        rms = jnp.sqrt(jnp.mean(base * base, axis=-1, keepdims=True) + 1e-6)
        norm = (base / rms) * g_ref[...]
        grp_buf[pl.ds(my_z * mz, mz), :] = norm.astype(jnp.bfloat16)
        out_ref[pl.ds(gb + my_z * mz, mz), :] = norm

        ag_rdmas = []
        for k in range(1, N_Z):
            r = (my_z - k) % N_Z
            rdma = pltpu.make_async_remote_copy(
                src_ref=grp_buf.at[pl.ds(my_z * mz, mz), :],
                dst_ref=grp_buf.at[pl.ds(my_z * mz, mz), :],
                send_sem=ag_send_sems.at[k - 1],
                recv_sem=ag_recv_sems.at[k - 1],
                device_id=(my_x, my_y, r),
                device_id_type=pl.DeviceIdType.MESH,
            )
            rdma.start()
            drains.append(rdma)
            ag_rdmas.append(rdma)

        for j in range(N_Z - 1):
            ag_rdmas[j].wait_recv()
            s = (my_z + j + 1) % N_Z
            out_ref[pl.ds(gb + s * mz, mz), :] = (
                grp_buf[pl.ds(s * mz, mz), :].astype(jnp.float32))

        yg_rdmas = []
        for k in range(1, N_Y):
            ty = (my_y - k) % N_Y
            rdma = pltpu.make_async_remote_copy(
                src_ref=grp_buf,
                dst_ref=yg_buf.at[k - 1],
                send_sem=yg_send_sems.at[k - 1],
                recv_sem=yg_recv_sems.at[k - 1],
                device_id=(my_x, ty, my_z),
                device_id_type=pl.DeviceIdType.MESH,
            )
            rdma.start()
            drains.append(rdma)
            yg_rdmas.append(rdma)

        def x_push(slot, src):
            rdma = pltpu.make_async_remote_copy(
                src_ref=src,
                dst_ref=xg_buf.at[slot],
                send_sem=x_send_sems.at[slot],
                recv_sem=x_recv_sems.at[slot],
                device_id=(xn, my_y, my_z),
                device_id_type=pl.DeviceIdType.MESH,
            )
            rdma.start()
            drains.append(rdma)
            return rdma

        x_rdmas = [x_push(0, grp_buf)]

        for j in range(N_Y - 1):
            yg_rdmas[j].wait_recv()
            x_rdmas.append(x_push(j + 1, yg_buf.at[j]))
            sy = (my_y + j + 1) % N_Y
            out_ref[pl.ds(my_x * mh + sy * mg, mg), :] = (
                yg_buf[j].astype(jnp.float32))

        x_rdmas[0].wait_recv()
        out_ref[pl.ds(xn * mh + my_y * mg, mg), :] = xg_buf[0].astype(jnp.float32)
        for j in range(N_Y - 1):
            x_rdmas[j + 1].wait_recv()
            sy = (my_y + j + 1) % N_Y
            out_ref[pl.ds(xn * mh + sy * mg, mg), :] = (
                xg_buf[j + 1].astype(jnp.float32))

        for rdma in drains:
            rdma.wait_send()

    return pl.pallas_call(
        body,
        out_shape=jax.ShapeDtypeStruct((m, d), jnp.float32),
        in_specs=[
            pl.BlockSpec(memory_space=pltpu.VMEM),
            pl.BlockSpec(memory_space=pltpu.VMEM),
            pl.BlockSpec(memory_space=pltpu.VMEM),
        ],
        out_specs=pl.BlockSpec(memory_space=pltpu.VMEM),
        scratch_shapes=[
            pltpu.VMEM((N_Z - 1, mz, d), jnp.bfloat16),
            pltpu.VMEM((N_Z - 1, mz, d), jnp.bfloat16),
            pltpu.VMEM((mg, d), jnp.bfloat16),
            pltpu.VMEM((N_Y - 1, mg, d), jnp.bfloat16),
            pltpu.VMEM((N_Y, mg, d), jnp.bfloat16),
            pltpu.SemaphoreType.DMA((N_Z - 1,)),
            pltpu.SemaphoreType.DMA((N_Z - 1,)),
            pltpu.SemaphoreType.DMA((N_Z - 1,)),
            pltpu.SemaphoreType.DMA((N_Z - 1,)),
            pltpu.SemaphoreType.DMA((N_Y - 1,)),
            pltpu.SemaphoreType.DMA((N_Y - 1,)),
            pltpu.SemaphoreType.DMA((N_Y,)),
            pltpu.SemaphoreType.DMA((N_Y,)),
        ],
        compiler_params=pltpu.CompilerParams(collective_id=0),
    )(partial, resid, gamma2)
